# baseline (speedup 1.0000x reference)
import functools

import jax
import jax.numpy as jnp
from jax import lax
from jax.experimental import pallas as pl
from jax.experimental.pallas import tpu as pltpu
from jax.experimental.pallas import tpu_sc as plsc

BATCH = 16384
HIST_LEN = 200

_INFO = plsc.get_sparse_core_info()
_NC, _NS = _INFO.num_cores, _INFO.num_subcores
_NW = _NC * _NS
_ROWS_PER_W = BATCH // _NW


@functools.partial(
    pl.kernel,
    out_type=jax.ShapeDtypeStruct((BATCH, HIST_LEN), jnp.float32),
    mesh=plsc.VectorSubcoreMesh(core_axis_name="c", subcore_axis_name="s"),
)
def _sc_copy(in_hbm, out_hbm):
    wid = lax.axis_index("s") * _NC + lax.axis_index("c")
    base = wid * _ROWS_PER_W
    pltpu.sync_copy(
        in_hbm.at[pl.ds(base, _ROWS_PER_W)],
        out_hbm.at[pl.ds(base, _ROWS_PER_W)],
    )


def kernel(inputs, embedding_table):
    del embedding_table
    return _sc_copy(inputs)


# speedup vs baseline: 1.0328x; 1.0328x over previous
import jax
import jax.numpy as jnp
from jax.experimental import pallas as pl
from jax.experimental.pallas import tpu as pltpu

BATCH = 16384
HIST_LEN = 200

_N_DMA = 8
_ROWS = BATCH // _N_DMA


def _copy_body(in_ref, out_ref, *sems):
    copies = [
        pltpu.make_async_copy(
            in_ref.at[pl.ds(i * _ROWS, _ROWS)],
            out_ref.at[pl.ds(i * _ROWS, _ROWS)],
            sems[i],
        )
        for i in range(_N_DMA)
    ]
    for c in copies:
        c.start()
    for c in copies:
        c.wait()


def kernel(inputs, embedding_table):
    del embedding_table
    return pl.pallas_call(
        _copy_body,
        out_shape=jax.ShapeDtypeStruct((BATCH, HIST_LEN), jnp.float32),
        in_specs=[pl.BlockSpec(memory_space=pltpu.MemorySpace.HBM)],
        out_specs=pl.BlockSpec(memory_space=pltpu.MemorySpace.HBM),
        scratch_shapes=[pltpu.SemaphoreType.DMA] * _N_DMA,
    )(inputs)


# speedup vs baseline: 12.9460x; 12.5342x over previous
import jax
import jax.numpy as jnp
from jax.experimental import pallas as pl
from jax.experimental.pallas import tpu as pltpu

BATCH = 16384
HIST_LEN = 200

_BLOCK_ROWS = 2048
_GRID = BATCH // _BLOCK_ROWS


def _copy_body(in_ref, out_ref):
    out_ref[...] = in_ref[...]


def kernel(inputs, embedding_table):
    del embedding_table
    return pl.pallas_call(
        _copy_body,
        out_shape=jax.ShapeDtypeStruct((BATCH, HIST_LEN), jnp.float32),
        grid=(_GRID,),
        in_specs=[pl.BlockSpec((_BLOCK_ROWS, HIST_LEN), lambda i: (i, 0))],
        out_specs=pl.BlockSpec((_BLOCK_ROWS, HIST_LEN), lambda i: (i, 0)),
        compiler_params=pltpu.CompilerParams(
            dimension_semantics=("arbitrary",),
        ),
    )(inputs)


# speedup vs baseline: 13.5669x; 1.0480x over previous
import jax
import jax.numpy as jnp
from jax.experimental import pallas as pl
from jax.experimental.pallas import tpu as pltpu

BATCH = 16384
HIST_LEN = 200

_N_CHUNK = 16
_ROWS = BATCH // _N_CHUNK


def _copy_body(in_ref, out_ref, *rest):
    bufs = rest[:_N_CHUNK]
    in_sems = rest[_N_CHUNK : 2 * _N_CHUNK]
    out_sems = rest[2 * _N_CHUNK :]
    ins = [
        pltpu.make_async_copy(
            in_ref.at[pl.ds(i * _ROWS, _ROWS)], bufs[i], in_sems[i]
        )
        for i in range(_N_CHUNK)
    ]
    outs = [
        pltpu.make_async_copy(
            bufs[i], out_ref.at[pl.ds(i * _ROWS, _ROWS)], out_sems[i]
        )
        for i in range(_N_CHUNK)
    ]
    for c in ins:
        c.start()
    for i in range(_N_CHUNK):
        ins[i].wait()
        outs[i].start()
    for c in outs:
        c.wait()


def kernel(inputs, embedding_table):
    del embedding_table
    return pl.pallas_call(
        _copy_body,
        out_shape=jax.ShapeDtypeStruct((BATCH, HIST_LEN), jnp.float32),
        in_specs=[pl.BlockSpec(memory_space=pltpu.MemorySpace.HBM)],
        out_specs=pl.BlockSpec(memory_space=pltpu.MemorySpace.HBM),
        scratch_shapes=(
            [pltpu.VMEM((_ROWS, HIST_LEN), jnp.float32)] * _N_CHUNK
            + [pltpu.SemaphoreType.DMA] * (2 * _N_CHUNK)
        ),
    )(inputs)
